# baseline (speedup 1.0000x reference)
import functools

import jax
from jax import lax
import jax.numpy as jnp
from jax.experimental import pallas as pl
from jax.experimental.pallas import tpu as pltpu
from jax.experimental.pallas import tpu_sc as plsc

KTOP = 2
TM = 512
IN = 1024

_SC_CORES = 2
_SC_SUBCORES = 16
_SC_NW = _SC_CORES * _SC_SUBCORES
_GCH = 32


def _sc_row_gather(table, idx):
    _, d = table.shape
    bt = idx.shape[0]
    b_per_w = bt // _SC_NW
    mesh = plsc.VectorSubcoreMesh(core_axis_name="c", subcore_axis_name="s")

    @functools.partial(
        pl.kernel, mesh=mesh,
        out_type=jax.ShapeDtypeStruct((bt, d), table.dtype),
        scratch_types=[
            pltpu.VMEM((b_per_w,), jnp.int32),
            pltpu.VMEM((_GCH, d), table.dtype),
            pltpu.VMEM((_GCH, d), table.dtype),
            pltpu.SemaphoreType.DMA,
            pltpu.SemaphoreType.DMA,
        ],
    )
    def k(table_hbm, idx_hbm, out_hbm, idx_v, buf0, buf1, sem0, sem1):
        wid = lax.axis_index("s") * _SC_CORES + lax.axis_index("c")
        base = wid * b_per_w
        pltpu.sync_copy(idx_hbm.at[pl.ds(base, b_per_w)], idx_v)

        @pl.loop(0, b_per_w // _GCH // 2)
        def _(o):
            c0 = o * 2
            c1 = c0 + 1
            cp0 = pltpu.async_copy(
                table_hbm.at[idx_v.at[pl.ds(c0 * _GCH, _GCH)]], buf0, sem0)
            cp1 = pltpu.async_copy(
                table_hbm.at[idx_v.at[pl.ds(c1 * _GCH, _GCH)]], buf1, sem1)
            cp0.wait()
            pltpu.sync_copy(buf0, out_hbm.at[pl.ds(base + c0 * _GCH, _GCH)])
            cp1.wait()
            pltpu.sync_copy(buf1, out_hbm.at[pl.ds(base + c1 * _GCH, _GCH)])

    return k(table, idx)


def _moe_mlp_kernel(gid_ref, nused_ref, xs_ref, wg_ref, w1_ref, w2_ref,
                    bg_ref, b1_ref, b2_ref, out_ref):
    n = pl.program_id(0)
    m = pl.program_id(1)

    @pl.when(m < nused_ref[0])
    def _():
        xb = xs_ref[...].astype(jnp.float32)
        g = jax.lax.dot_general(xb, wg_ref[0], (((1,), (0,)), ((), ())),
                                preferred_element_type=jnp.float32)
        u = jax.lax.dot_general(xb, w1_ref[0], (((1,), (0,)), ((), ())),
                                preferred_element_type=jnp.float32)
        g = g + bg_ref[0]
        u = u + b1_ref[0]
        gelu = g * 0.5 * (1.0 + jax.lax.erf(g * 0.7071067811865476))
        h = gelu * u
        y = jax.lax.dot_general(h, w2_ref[0], (((1,), (0,)), ((), ())),
                                preferred_element_type=jnp.float32)

        @pl.when(n == 0)
        def _():
            out_ref[0] = y + b2_ref[0]

        @pl.when(n > 0)
        def _():
            out_ref[0] = y


def _grouped_mlp(xs, gids, nused, Wg, bg, W1, b1, W2, b2):
    m_pad, d = xs.shape
    e, _, i = Wg.shape
    m_tiles = m_pad // TM
    n_chunks = i // IN

    grid_spec = pltpu.PrefetchScalarGridSpec(
        num_scalar_prefetch=2,
        grid=(n_chunks, m_tiles),
        in_specs=[
            pl.BlockSpec((TM, d), lambda n, m, g, nu: (m, 0)),
            pl.BlockSpec((1, d, IN), lambda n, m, g, nu: (g[m], 0, n)),
            pl.BlockSpec((1, d, IN), lambda n, m, g, nu: (g[m], 0, n)),
            pl.BlockSpec((1, IN, d), lambda n, m, g, nu: (g[m], n, 0)),
            pl.BlockSpec((1, 1, IN), lambda n, m, g, nu: (g[m], 0, n)),
            pl.BlockSpec((1, 1, IN), lambda n, m, g, nu: (g[m], 0, n)),
            pl.BlockSpec((1, 1, d), lambda n, m, g, nu: (g[m], 0, 0)),
        ],
        out_specs=pl.BlockSpec((1, TM, d), lambda n, m, g, nu: (n, m, 0)),
    )
    return pl.pallas_call(
        _moe_mlp_kernel,
        grid_spec=grid_spec,
        out_shape=jax.ShapeDtypeStruct((n_chunks, m_pad, d), jnp.float32),
    )(gids, nused, xs, Wg, W1, W2,
      bg[:, None, :], b1[:, None, :], b2[:, None, :])


def kernel(x, Wr, Wg, bg, W1, b1, W2, b2):
    b, s, d = x.shape
    e = Wg.shape[0]
    n_tok = b * s
    n_asn = n_tok * KTOP
    m_pad = n_asn + e * TM
    m_tiles = m_pad // TM

    logits = x @ Wr
    probs = jax.nn.softmax(logits, axis=-1).reshape(n_tok, e)
    i1 = jnp.argmax(probs, axis=-1).astype(jnp.int32)
    p1 = jnp.max(probs, axis=-1)
    lane = jnp.arange(e, dtype=jnp.int32)[None, :]
    masked = jnp.where(lane == i1[:, None], -jnp.inf, probs)
    i2 = jnp.argmax(masked, axis=-1).astype(jnp.int32)
    p2 = jnp.max(masked, axis=-1)
    w = jnp.stack([p1, p2], axis=1)
    e_idx = jnp.stack([i1, i2], axis=1)

    e_flat = e_idx.reshape(-1)
    onehot = (e_flat[:, None] == jnp.arange(e, dtype=jnp.int32)[None, :])
    onehot = onehot.astype(jnp.int32)
    incl = jnp.cumsum(onehot, axis=0)
    rank = jnp.take_along_axis(incl - onehot, e_flat[:, None], axis=1)[:, 0]
    counts = incl[-1]
    tiles_per = (counts + TM - 1) // TM
    tile_off = jnp.concatenate([jnp.zeros((1,), jnp.int32),
                                jnp.cumsum(tiles_per)]).astype(jnp.int32)
    nused = tile_off[e:e + 1]
    pos = tile_off[e_flat] * TM + rank
    tid = jnp.arange(m_tiles, dtype=jnp.int32)
    gids = jnp.sum((tid[:, None] >= tile_off[None, 1:]).astype(jnp.int32),
                   axis=1)
    gids = jnp.minimum(gids, e - 1)

    xf = x.reshape(n_tok, d).astype(jnp.bfloat16)
    aid = jnp.arange(n_asn, dtype=jnp.int32) // KTOP
    sorted_tid = jnp.zeros((m_pad,), jnp.int32).at[pos].set(aid)
    xs = jnp.take(xf, sorted_tid, axis=0)

    ysn = _grouped_mlp(xs, gids, nused, Wg, bg, W1, b1, W2, b2)
    ys = ysn.sum(axis=0)

    sel = _sc_row_gather(ys, pos).reshape(n_tok, KTOP, d)
    out = jnp.sum(sel * w[:, :, None], axis=1)
    return out.reshape(b, s, d)


# speedup vs baseline: 1.0138x; 1.0138x over previous
import functools

import jax
from jax import lax
import jax.numpy as jnp
from jax.experimental import pallas as pl
from jax.experimental.pallas import tpu as pltpu
from jax.experimental.pallas import tpu_sc as plsc

KTOP = 2
TM = 256
IN = 2048

_SC_CORES = 2
_SC_SUBCORES = 16
_SC_NW = _SC_CORES * _SC_SUBCORES
_GCH = 32


def _sc_row_gather(table, idx):
    _, d = table.shape
    bt = idx.shape[0]
    b_per_w = bt // _SC_NW
    mesh = plsc.VectorSubcoreMesh(core_axis_name="c", subcore_axis_name="s")

    @functools.partial(
        pl.kernel, mesh=mesh,
        out_type=jax.ShapeDtypeStruct((bt, d), table.dtype),
        scratch_types=[
            pltpu.VMEM((b_per_w,), jnp.int32),
            pltpu.VMEM((_GCH, d), table.dtype),
            pltpu.VMEM((_GCH, d), table.dtype),
            pltpu.SemaphoreType.DMA,
            pltpu.SemaphoreType.DMA,
        ],
    )
    def k(table_hbm, idx_hbm, out_hbm, idx_v, buf0, buf1, sem0, sem1):
        wid = lax.axis_index("s") * _SC_CORES + lax.axis_index("c")
        base = wid * b_per_w
        pltpu.sync_copy(idx_hbm.at[pl.ds(base, b_per_w)], idx_v)

        @pl.loop(0, b_per_w // _GCH // 2)
        def _(o):
            c0 = o * 2
            c1 = c0 + 1
            cp0 = pltpu.async_copy(
                table_hbm.at[idx_v.at[pl.ds(c0 * _GCH, _GCH)]], buf0, sem0)
            cp1 = pltpu.async_copy(
                table_hbm.at[idx_v.at[pl.ds(c1 * _GCH, _GCH)]], buf1, sem1)
            cp0.wait()
            pltpu.sync_copy(buf0, out_hbm.at[pl.ds(base + c0 * _GCH, _GCH)])
            cp1.wait()
            pltpu.sync_copy(buf1, out_hbm.at[pl.ds(base + c1 * _GCH, _GCH)])

    return k(table, idx)


def _moe_mlp_kernel(gid_ref, nused_ref, xs_ref, wg_ref, w1_ref, w2_ref,
                    bg_ref, b1_ref, b2_ref, out_ref):
    n = pl.program_id(0)
    m = pl.program_id(1)

    @pl.when(m < nused_ref[0])
    def _():
        xb = xs_ref[...].astype(jnp.float32)
        g = jax.lax.dot_general(xb, wg_ref[0], (((1,), (0,)), ((), ())),
                                preferred_element_type=jnp.float32)
        u = jax.lax.dot_general(xb, w1_ref[0], (((1,), (0,)), ((), ())),
                                preferred_element_type=jnp.float32)
        g = g + bg_ref[0]
        u = u + b1_ref[0]
        gelu = g * 0.5 * (1.0 + jax.lax.erf(g * 0.7071067811865476))
        h = gelu * u
        y = jax.lax.dot_general(h, w2_ref[0], (((1,), (0,)), ((), ())),
                                preferred_element_type=jnp.float32)

        @pl.when(n == 0)
        def _():
            out_ref[0] = y + b2_ref[0]

        @pl.when(n > 0)
        def _():
            out_ref[0] = y


def _grouped_mlp(xs, gids, nused, Wg, bg, W1, b1, W2, b2):
    m_pad, d = xs.shape
    e, _, i = Wg.shape
    m_tiles = m_pad // TM
    n_chunks = i // IN

    grid_spec = pltpu.PrefetchScalarGridSpec(
        num_scalar_prefetch=2,
        grid=(n_chunks, m_tiles),
        in_specs=[
            pl.BlockSpec((TM, d), lambda n, m, g, nu: (m, 0)),
            pl.BlockSpec((1, d, IN), lambda n, m, g, nu: (g[m], 0, n)),
            pl.BlockSpec((1, d, IN), lambda n, m, g, nu: (g[m], 0, n)),
            pl.BlockSpec((1, IN, d), lambda n, m, g, nu: (g[m], n, 0)),
            pl.BlockSpec((1, 1, IN), lambda n, m, g, nu: (g[m], 0, n)),
            pl.BlockSpec((1, 1, IN), lambda n, m, g, nu: (g[m], 0, n)),
            pl.BlockSpec((1, 1, d), lambda n, m, g, nu: (g[m], 0, 0)),
        ],
        out_specs=pl.BlockSpec((1, TM, d), lambda n, m, g, nu: (n, m, 0)),
    )
    return pl.pallas_call(
        _moe_mlp_kernel,
        grid_spec=grid_spec,
        out_shape=jax.ShapeDtypeStruct((n_chunks, m_pad, d), jnp.float32),
    )(gids, nused, xs, Wg, W1, W2,
      bg[:, None, :], b1[:, None, :], b2[:, None, :])


def kernel(x, Wr, Wg, bg, W1, b1, W2, b2):
    b, s, d = x.shape
    e = Wg.shape[0]
    n_tok = b * s
    n_asn = n_tok * KTOP
    m_pad = n_asn + e * TM
    m_tiles = m_pad // TM

    logits = x @ Wr
    probs = jax.nn.softmax(logits, axis=-1).reshape(n_tok, e)
    i1 = jnp.argmax(probs, axis=-1).astype(jnp.int32)
    p1 = jnp.max(probs, axis=-1)
    lane = jnp.arange(e, dtype=jnp.int32)[None, :]
    masked = jnp.where(lane == i1[:, None], -jnp.inf, probs)
    i2 = jnp.argmax(masked, axis=-1).astype(jnp.int32)
    p2 = jnp.max(masked, axis=-1)
    w = jnp.stack([p1, p2], axis=1)
    e_idx = jnp.stack([i1, i2], axis=1)

    e_flat = e_idx.reshape(-1)
    onehot = (e_flat[:, None] == jnp.arange(e, dtype=jnp.int32)[None, :])
    onehot = onehot.astype(jnp.int32)
    incl = jnp.cumsum(onehot, axis=0)
    rank = jnp.take_along_axis(incl - onehot, e_flat[:, None], axis=1)[:, 0]
    counts = incl[-1]
    tiles_per = (counts + TM - 1) // TM
    tile_off = jnp.concatenate([jnp.zeros((1,), jnp.int32),
                                jnp.cumsum(tiles_per)]).astype(jnp.int32)
    nused = tile_off[e:e + 1]
    pos = tile_off[e_flat] * TM + rank
    tid = jnp.arange(m_tiles, dtype=jnp.int32)
    gids = jnp.sum((tid[:, None] >= tile_off[None, 1:]).astype(jnp.int32),
                   axis=1)
    gids = jnp.minimum(gids, e - 1)

    xf = x.reshape(n_tok, d).astype(jnp.bfloat16)
    aid = jnp.arange(n_asn, dtype=jnp.int32) // KTOP
    sorted_tid = jnp.zeros((m_pad,), jnp.int32).at[pos].set(aid)
    xs = jnp.take(xf, sorted_tid, axis=0)

    ysn = _grouped_mlp(xs, gids, nused, Wg, bg, W1, b1, W2, b2)

    n_chunks = ysn.shape[0]
    pos2 = jnp.concatenate([pos + c * m_pad for c in range(n_chunks)])
    sel = _sc_row_gather(ysn.reshape(n_chunks * m_pad, d), pos2)
    sel = sel.reshape(n_chunks, n_tok, KTOP, d)
    out = jnp.sum(sel.sum(axis=0) * w[:, :, None], axis=1)
    return out.reshape(b, s, d)


# speedup vs baseline: 1.1115x; 1.0964x over previous
import functools

import jax
from jax import lax
import jax.numpy as jnp
from jax.experimental import pallas as pl
from jax.experimental.pallas import tpu as pltpu
from jax.experimental.pallas import tpu_sc as plsc

KTOP = 2
TM = 256
IN = 2048

_SC_CORES = 2
_SC_SUBCORES = 16
_SC_NW = _SC_CORES * _SC_SUBCORES
_GCH = 32


def _sc_row_gather(table, idx):
    _, d = table.shape
    bt = idx.shape[0]
    b_per_w = bt // _SC_NW
    mesh = plsc.VectorSubcoreMesh(core_axis_name="c", subcore_axis_name="s")

    @functools.partial(
        pl.kernel, mesh=mesh,
        out_type=jax.ShapeDtypeStruct((bt, d), table.dtype),
        scratch_types=[
            pltpu.VMEM((b_per_w,), jnp.int32),
            pltpu.VMEM((_GCH, d), table.dtype),
            pltpu.VMEM((_GCH, d), table.dtype),
            pltpu.SemaphoreType.DMA,
            pltpu.SemaphoreType.DMA,
        ],
    )
    def k(table_hbm, idx_hbm, out_hbm, idx_v, buf0, buf1, sem0, sem1):
        wid = lax.axis_index("s") * _SC_CORES + lax.axis_index("c")
        base = wid * b_per_w
        pltpu.sync_copy(idx_hbm.at[pl.ds(base, b_per_w)], idx_v)

        @pl.loop(0, b_per_w // _GCH // 2)
        def _(o):
            c0 = o * 2
            c1 = c0 + 1
            cp0 = pltpu.async_copy(
                table_hbm.at[idx_v.at[pl.ds(c0 * _GCH, _GCH)]], buf0, sem0)
            cp1 = pltpu.async_copy(
                table_hbm.at[idx_v.at[pl.ds(c1 * _GCH, _GCH)]], buf1, sem1)
            cp0.wait()
            pltpu.sync_copy(buf0, out_hbm.at[pl.ds(base + c0 * _GCH, _GCH)])
            cp1.wait()
            pltpu.sync_copy(buf1, out_hbm.at[pl.ds(base + c1 * _GCH, _GCH)])

    return k(table, idx)


def _moe_mlp_kernel(gid_ref, nused_ref, xs_ref, wg_ref, w1_ref, w2_ref,
                    bg_ref, b1_ref, b2_ref, out_ref):
    n = pl.program_id(0)
    m = pl.program_id(1)

    @pl.when(m < nused_ref[0])
    def _():
        xb = xs_ref[...].astype(jnp.float32)
        g = jax.lax.dot_general(xb, wg_ref[0], (((1,), (0,)), ((), ())),
                                preferred_element_type=jnp.float32)
        u = jax.lax.dot_general(xb, w1_ref[0], (((1,), (0,)), ((), ())),
                                preferred_element_type=jnp.float32)
        g = g + bg_ref[0]
        u = u + b1_ref[0]
        gelu = g * 0.5 * (1.0 + jax.lax.erf(g * 0.7071067811865476))
        h = gelu * u
        y = jax.lax.dot_general(h, w2_ref[0], (((1,), (0,)), ((), ())),
                                preferred_element_type=jnp.float32)

        @pl.when(n == 0)
        def _():
            out_ref[0] = y + b2_ref[0]

        @pl.when(n > 0)
        def _():
            out_ref[0] = y


def _grouped_mlp(xs, gids, nused, Wg, bg, W1, b1, W2, b2):
    m_pad, d = xs.shape
    e, _, i = Wg.shape
    m_tiles = m_pad // TM
    n_chunks = i // IN

    grid_spec = pltpu.PrefetchScalarGridSpec(
        num_scalar_prefetch=2,
        grid=(n_chunks, m_tiles),
        in_specs=[
            pl.BlockSpec((TM, d), lambda n, m, g, nu: (m, 0)),
            pl.BlockSpec((1, d, IN), lambda n, m, g, nu: (g[m], 0, n)),
            pl.BlockSpec((1, d, IN), lambda n, m, g, nu: (g[m], 0, n)),
            pl.BlockSpec((1, IN, d), lambda n, m, g, nu: (g[m], n, 0)),
            pl.BlockSpec((1, 1, IN), lambda n, m, g, nu: (g[m], 0, n)),
            pl.BlockSpec((1, 1, IN), lambda n, m, g, nu: (g[m], 0, n)),
            pl.BlockSpec((1, 1, d), lambda n, m, g, nu: (g[m], 0, 0)),
        ],
        out_specs=pl.BlockSpec((1, TM, d), lambda n, m, g, nu: (n, m, 0)),
    )
    return pl.pallas_call(
        _moe_mlp_kernel,
        grid_spec=grid_spec,
        out_shape=jax.ShapeDtypeStruct((n_chunks, m_pad, d), jnp.float32),
    )(gids, nused, xs, Wg, W1, W2,
      bg[:, None, :], b1[:, None, :], b2[:, None, :])


def kernel(x, Wr, Wg, bg, W1, b1, W2, b2):
    b, s, d = x.shape
    e = Wg.shape[0]
    n_tok = b * s
    n_asn = n_tok * KTOP
    m_pad = n_asn + e * TM
    m_tiles = m_pad // TM

    logits = x @ Wr
    probs = jax.nn.softmax(logits, axis=-1).reshape(n_tok, e)
    i1 = jnp.argmax(probs, axis=-1).astype(jnp.int32)
    p1 = jnp.max(probs, axis=-1)
    lane = jnp.arange(e, dtype=jnp.int32)[None, :]
    masked = jnp.where(lane == i1[:, None], -jnp.inf, probs)
    i2 = jnp.argmax(masked, axis=-1).astype(jnp.int32)
    p2 = jnp.max(masked, axis=-1)
    w = jnp.stack([p1, p2], axis=1)
    e_idx = jnp.stack([i1, i2], axis=1)

    e_flat = e_idx.reshape(-1)
    onehot = (e_flat[:, None] == jnp.arange(e, dtype=jnp.int32)[None, :])
    onehot = onehot.astype(jnp.int32)
    incl = jnp.cumsum(onehot, axis=0)
    rank = jnp.take_along_axis(incl - onehot, e_flat[:, None], axis=1)[:, 0]
    counts = incl[-1]
    tiles_per = (counts + TM - 1) // TM
    tile_off = jnp.concatenate([jnp.zeros((1,), jnp.int32),
                                jnp.cumsum(tiles_per)]).astype(jnp.int32)
    nused = tile_off[e:e + 1]
    pos = tile_off[e_flat] * TM + rank
    tid = jnp.arange(m_tiles, dtype=jnp.int32)
    gids = jnp.sum((tid[:, None] >= tile_off[None, 1:]).astype(jnp.int32),
                   axis=1)
    gids = jnp.minimum(gids, e - 1)

    xf = x.reshape(n_tok, d).astype(jnp.bfloat16)
    aid = jnp.arange(n_asn, dtype=jnp.int32) // KTOP
    sorted_tid = jnp.zeros((m_pad,), jnp.int32).at[pos].set(aid)
    xs = jnp.take(xf, sorted_tid, axis=0)

    ysn = _grouped_mlp(xs, gids, nused, Wg, bg, W1, b1, W2, b2)
    ys = ysn.sum(axis=0)

    sel = _sc_row_gather(ys, pos).reshape(n_tok, KTOP, d)
    out = jnp.sum(sel * w[:, :, None], axis=1)
    return out.reshape(b, s, d)


# speedup vs baseline: 1.1365x; 1.0224x over previous
import functools

import jax
from jax import lax
import jax.numpy as jnp
from jax.experimental import pallas as pl
from jax.experimental.pallas import tpu as pltpu
from jax.experimental.pallas import tpu_sc as plsc

KTOP = 2
TM = 256
IN = 2048

_SC_CORES = 2
_SC_SUBCORES = 16
_SC_NW = _SC_CORES * _SC_SUBCORES
_GCH = 32


def _sc_row_gather(table, idx):
    _, d = table.shape
    bt = idx.shape[0]
    b_per_w = bt // _SC_NW
    mesh = plsc.VectorSubcoreMesh(core_axis_name="c", subcore_axis_name="s")

    @functools.partial(
        pl.kernel, mesh=mesh,
        out_type=jax.ShapeDtypeStruct((bt, d), table.dtype),
        scratch_types=[
            pltpu.VMEM((b_per_w,), jnp.int32),
            pltpu.VMEM((_GCH, d), table.dtype),
            pltpu.VMEM((_GCH, d), table.dtype),
            pltpu.SemaphoreType.DMA,
            pltpu.SemaphoreType.DMA,
        ],
    )
    def k(table_hbm, idx_hbm, out_hbm, idx_v, buf0, buf1, sem0, sem1):
        wid = lax.axis_index("s") * _SC_CORES + lax.axis_index("c")
        base = wid * b_per_w
        pltpu.sync_copy(idx_hbm.at[pl.ds(base, b_per_w)], idx_v)

        @pl.loop(0, b_per_w // _GCH // 2)
        def _(o):
            c0 = o * 2
            c1 = c0 + 1
            cp0 = pltpu.async_copy(
                table_hbm.at[idx_v.at[pl.ds(c0 * _GCH, _GCH)]], buf0, sem0)
            cp1 = pltpu.async_copy(
                table_hbm.at[idx_v.at[pl.ds(c1 * _GCH, _GCH)]], buf1, sem1)
            cp0.wait()
            pltpu.sync_copy(buf0, out_hbm.at[pl.ds(base + c0 * _GCH, _GCH)])
            cp1.wait()
            pltpu.sync_copy(buf1, out_hbm.at[pl.ds(base + c1 * _GCH, _GCH)])

    return k(table, idx)


def _moe_mlp_kernel(gid_ref, nused_ref, xs_ref, wg_ref, w1_ref, w2_ref,
                    bg_ref, b1_ref, b2_ref, out_ref):
    n = pl.program_id(0)
    m = pl.program_id(1)

    @pl.when(m < nused_ref[0])
    def _():
        xb = xs_ref[...].astype(jnp.float32)
        g = jax.lax.dot_general(xb, wg_ref[0], (((1,), (0,)), ((), ())),
                                preferred_element_type=jnp.float32)
        u = jax.lax.dot_general(xb, w1_ref[0], (((1,), (0,)), ((), ())),
                                preferred_element_type=jnp.float32)
        g = g + bg_ref[0]
        u = u + b1_ref[0]
        gelu = g * 0.5 * (1.0 + jax.lax.erf(g * 0.7071067811865476))
        h = gelu * u
        y = jax.lax.dot_general(h, w2_ref[0], (((1,), (0,)), ((), ())),
                                preferred_element_type=jnp.float32)

        @pl.when(n == 0)
        def _():
            out_ref[0] = (y + b2_ref[0]).astype(jnp.bfloat16)

        @pl.when(n > 0)
        def _():
            out_ref[0] = y.astype(jnp.bfloat16)


def _grouped_mlp(xs, gids, nused, Wg, bg, W1, b1, W2, b2):
    m_pad, d = xs.shape
    e, _, i = Wg.shape
    m_tiles = m_pad // TM
    n_chunks = i // IN

    grid_spec = pltpu.PrefetchScalarGridSpec(
        num_scalar_prefetch=2,
        grid=(n_chunks, m_tiles),
        in_specs=[
            pl.BlockSpec((TM, d), lambda n, m, g, nu: (m, 0)),
            pl.BlockSpec((1, d, IN), lambda n, m, g, nu: (g[m], 0, n)),
            pl.BlockSpec((1, d, IN), lambda n, m, g, nu: (g[m], 0, n)),
            pl.BlockSpec((1, IN, d), lambda n, m, g, nu: (g[m], n, 0)),
            pl.BlockSpec((1, 1, IN), lambda n, m, g, nu: (g[m], 0, n)),
            pl.BlockSpec((1, 1, IN), lambda n, m, g, nu: (g[m], 0, n)),
            pl.BlockSpec((1, 1, d), lambda n, m, g, nu: (g[m], 0, 0)),
        ],
        out_specs=pl.BlockSpec((1, TM, d), lambda n, m, g, nu: (n, m, 0)),
    )
    return pl.pallas_call(
        _moe_mlp_kernel,
        grid_spec=grid_spec,
        out_shape=jax.ShapeDtypeStruct((n_chunks, m_pad, d), jnp.bfloat16),
    )(gids, nused, xs, Wg, W1, W2,
      bg[:, None, :], b1[:, None, :], b2[:, None, :])


def kernel(x, Wr, Wg, bg, W1, b1, W2, b2):
    b, s, d = x.shape
    e = Wg.shape[0]
    n_tok = b * s
    n_asn = n_tok * KTOP
    m_pad = n_asn + e * TM
    m_tiles = m_pad // TM

    logits = x @ Wr
    probs = jax.nn.softmax(logits, axis=-1).reshape(n_tok, e)
    i1 = jnp.argmax(probs, axis=-1).astype(jnp.int32)
    p1 = jnp.max(probs, axis=-1)
    lane = jnp.arange(e, dtype=jnp.int32)[None, :]
    masked = jnp.where(lane == i1[:, None], -jnp.inf, probs)
    i2 = jnp.argmax(masked, axis=-1).astype(jnp.int32)
    p2 = jnp.max(masked, axis=-1)
    w = jnp.stack([p1, p2], axis=1)
    e_idx = jnp.stack([i1, i2], axis=1)

    e_flat = e_idx.reshape(-1)
    onehot = (e_flat[:, None] == jnp.arange(e, dtype=jnp.int32)[None, :])
    onehot = onehot.astype(jnp.int32)
    incl = jnp.cumsum(onehot, axis=0)
    rank = jnp.take_along_axis(incl - onehot, e_flat[:, None], axis=1)[:, 0]
    counts = incl[-1]
    tiles_per = (counts + TM - 1) // TM
    tile_off = jnp.concatenate([jnp.zeros((1,), jnp.int32),
                                jnp.cumsum(tiles_per)]).astype(jnp.int32)
    nused = tile_off[e:e + 1]
    pos = tile_off[e_flat] * TM + rank
    tid = jnp.arange(m_tiles, dtype=jnp.int32)
    gids = jnp.sum((tid[:, None] >= tile_off[None, 1:]).astype(jnp.int32),
                   axis=1)
    gids = jnp.minimum(gids, e - 1)

    xf = x.reshape(n_tok, d).astype(jnp.bfloat16)
    aid = jnp.arange(n_asn, dtype=jnp.int32) // KTOP
    sorted_tid = jnp.zeros((m_pad,), jnp.int32).at[pos].set(aid)
    xs = jnp.take(xf, sorted_tid, axis=0)

    ysn = _grouped_mlp(xs, gids, nused, Wg, bg, W1, b1, W2, b2)
    ys = ysn.astype(jnp.float32).sum(axis=0)

    sel = _sc_row_gather(ys, pos).reshape(n_tok, KTOP, d)
    out = jnp.sum(sel * w[:, :, None], axis=1)
    return out.reshape(b, s, d)
